# baseline (speedup 1.0000x reference)
import functools

import jax
import jax.numpy as jnp
from jax import lax
from jax.experimental import pallas as pl
from jax.experimental.pallas import tpu as pltpu
from jax.experimental.pallas import tpu_sc as plsc

_N = 10000
_E = 320000
_NW = 32
_C = 80
_CHUNKS = (_E // _NW) // _C
_RPT = 640
_ZR = 80


_NG = 3
_NS = 2
_UNROLL = 6


def _ring_chunks(CH, gi, gw, comp, si, sw, prime=True):
  if prime:
    for b in range(_NG):
      gi(b, b)

  def step(j, t, static):
    g, s = t % _NG, t % _NS
    gw(j, g)
    if static:
      if j >= _NS:
        sw(j - _NS, s)
    else:
      @pl.when(j >= _NS)
      def _drain_prev():
        sw(j - _NS, s)
    comp(j, g, s)
    if static:
      if j + _NG < CH:
        gi(j + _NG, g)
    else:
      @pl.when(j + _NG < CH)
      def _prefetch():
        gi(j + _NG, g)
    si(j, s)

  P = CH // _UNROLL

  def ring(i, carry):
    for t in range(_UNROLL):
      step(i * _UNROLL + t, t, False)
    return carry
  lax.fori_loop(0, P, ring, None)
  for j in range(P * _UNROLL, CH):
    step(j, j % _UNROLL, True)
  sw(CH - 2, (CH - 2) % _NS)
  sw(CH - 1, (CH - 1) % _NS)


def _make_sc_colsplit():
  NL = 4
  CH = _E // 16 // _C
  mesh = plsc.VectorSubcoreMesh(core_axis_name="c", subcore_axis_name="s")

  @functools.partial(
      pl.kernel,
      mesh=mesh,
      compiler_params=pltpu.CompilerParams(use_tc_tiling_on_sc=False),
      out_type=jax.ShapeDtypeStruct((2, _N, 64), jnp.float32),
      scratch_types=[
          pltpu.VMEM((CH, _C), jnp.int32),
          pltpu.VMEM((CH, _C), jnp.int32),
          pltpu.VMEM((CH, _C), jnp.float32),
          pltpu.VMEM((_NG, _C, 64), jnp.float32),
          pltpu.VMEM((_NS, _C, 64), jnp.float32),
          pltpu.VMEM((128,), jnp.float32),
          pltpu.VMEM((128,), jnp.float32),
          pltpu.VMEM((_ZR, 64), jnp.float32),
          pltpu.VMEM_SHARED((_N, 64), jnp.float32),
          pltpu.SemaphoreType.DMA,
          pltpu.SemaphoreType.DMA,
          pltpu.SemaphoreType.DMA,
          pltpu.SemaphoreType.DMA,
          pltpu.SemaphoreType.DMA,
          pltpu.SemaphoreType.DMA,
          pltpu.SemaphoreType.DMA,
          pltpu.SemaphoreType.DMA,
      ],
  )
  def sc_l1(x2_hbm, srclo_hbm, srchi_hbm, dst_hbm, ea_hbm, w_hbm, b_hbm,
            out_hbm, src_v, dst_v, ea_v, gbuf_v, sbuf_v, w_v, b_v, zb_v,
            table, gsem0, gsem1, gsem2, ssem0, ssem1, isem, esem, zsem):
    c = lax.axis_index("c")
    s = lax.axis_index("s")

    @pl.when(c == 0)
    def _stage_lo():
      pltpu.async_copy(srclo_hbm.at[s], src_v, isem)

    @pl.when(c == 1)
    def _stage_hi():
      pltpu.async_copy(srchi_hbm.at[s], src_v, isem)
    dst_cp = pltpu.async_copy(dst_hbm.at[s], dst_v, esem)
    ea_cp = pltpu.async_copy(ea_hbm.at[s], ea_v, esem)
    pltpu.sync_copy(w_hbm, w_v)
    pltpu.sync_copy(b_hbm, b_v)
    pltpu.make_async_copy(srclo_hbm.at[s], src_v, isem).wait()

    wr = [w_v[pl.ds(c * 64 + 16 * i, 16)] for i in range(NL)]
    br = [b_v[pl.ds(c * 64 + 16 * i, 16)] for i in range(NL)]

    gsem = [gsem0, gsem1, gsem2]
    ssem = [ssem0, ssem1]

    def gi(j, b):
      pltpu.async_copy(x2_hbm.at[src_v.at[j]], gbuf_v.at[b], gsem[b])

    def gw(j, b):
      pltpu.make_async_copy(x2_hbm.at[src_v.at[j]], gbuf_v.at[b],
                            gsem[b]).wait()

    def si(j, b):
      pltpu.async_copy(sbuf_v.at[b], table.at[dst_v.at[j]], ssem[b],
                       add=True)

    def sw(j, b):
      pltpu.make_async_copy(sbuf_v.at[b], table.at[dst_v.at[j]],
                            ssem[b]).wait()

    def comp(j, gb, sb):
      def do_group(g):
        ev = ea_v[j, pl.ds(16 * g, 16)]
        for t in range(16):
          e = g * 16 + t
          ea = ev[t]
          for i in range(NL):
            r = gbuf_v[gb, e, pl.ds(16 * i, 16)]
            sbuf_v[sb, e, pl.ds(16 * i, 16)] = jnp.maximum(
                r + ea * wr[i] + br[i], 0.0)

      def egroup2(g, inner):
        do_group(2 * g)
        do_group(2 * g + 1)
        return inner
      lax.fori_loop(0, _C // 32, egroup2, None)
      do_group(_C // 16 - 1)

    for b in range(_NG):
      gi(b, b)

    def zrow(r, carry):
      for i in range(NL):
        zb_v[r, pl.ds(16 * i, 16)] = jnp.zeros((16,), jnp.float32)
      return carry
    lax.fori_loop(0, _ZR, zrow, None)
    nz = jnp.where(s == 15, 5, 8)

    def zcopy(k, carry):
      pltpu.async_copy(zb_v, table.at[pl.ds(s * _RPT + k * _ZR, _ZR)], zsem)
      return carry
    lax.fori_loop(0, nz, zcopy, None)

    def zdrain(k, carry):
      pltpu.make_async_copy(
          zb_v, table.at[pl.ds(s * _RPT + k * _ZR, _ZR)], zsem).wait()
      return carry
    lax.fori_loop(0, nz, zdrain, None)
    dst_cp.wait()
    ea_cp.wait()
    plsc.subcore_barrier()

    _ring_chunks(CH, gi, gw, comp, si, sw, prime=False)
    plsc.subcore_barrier()

    @pl.when(s < 15)
    def _dump_full():
      pltpu.sync_copy(table.at[pl.ds(s * _RPT, _RPT)],
                      out_hbm.at[c, pl.ds(s * _RPT, _RPT)])

    @pl.when(s == 15)
    def _dump_tail():
      pltpu.sync_copy(table.at[pl.ds(15 * _RPT, _N - 15 * _RPT)],
                      out_hbm.at[c, pl.ds(15 * _RPT, _N - 15 * _RPT)])

  return sc_l1


def _make_sc_layer(F):
  NL = F // 16
  mesh = plsc.VectorSubcoreMesh(core_axis_name="c", subcore_axis_name="s")

  @functools.partial(
      pl.kernel,
      mesh=mesh,
      compiler_params=pltpu.CompilerParams(use_tc_tiling_on_sc=False),
      out_type=jax.ShapeDtypeStruct((2, _N, F), jnp.float32),
      scratch_types=[
          pltpu.VMEM((_CHUNKS, _C), jnp.int32),
          pltpu.VMEM((_CHUNKS, _C), jnp.int32),
          pltpu.VMEM((_CHUNKS, _C), jnp.float32),
          pltpu.VMEM((_NG, _C, F), jnp.float32),
          pltpu.VMEM((_NS, _C, F), jnp.float32),
          pltpu.VMEM((F,), jnp.float32),
          pltpu.VMEM((F,), jnp.float32),
          pltpu.VMEM((_ZR, F), jnp.float32),
          pltpu.VMEM_SHARED((_N, F), jnp.float32),
          pltpu.SemaphoreType.DMA,
          pltpu.SemaphoreType.DMA,
          pltpu.SemaphoreType.DMA,
          pltpu.SemaphoreType.DMA,
          pltpu.SemaphoreType.DMA,
          pltpu.SemaphoreType.DMA,
          pltpu.SemaphoreType.DMA,
          pltpu.SemaphoreType.DMA,
      ],
  )
  def sc_layer(x_hbm, src_hbm, dst_hbm, ea_hbm, w_hbm, b_hbm, out_hbm,
               src_v, dst_v, ea_v, gbuf_v, sbuf_v, w_v, b_v, zb_v, table,
               gsem0, gsem1, gsem2, ssem0, ssem1, isem, esem, zsem):
    c = lax.axis_index("c")
    s = lax.axis_index("s")
    wid = c * 16 + s

    src_cp = pltpu.async_copy(src_hbm.at[wid], src_v, isem)
    dst_cp = pltpu.async_copy(dst_hbm.at[wid], dst_v, esem)
    ea_cp = pltpu.async_copy(ea_hbm.at[wid], ea_v, esem)
    pltpu.sync_copy(w_hbm, w_v)
    pltpu.sync_copy(b_hbm, b_v)
    src_cp.wait()

    wr = [w_v[pl.ds(16 * i, 16)] for i in range(NL)]
    br = [b_v[pl.ds(16 * i, 16)] for i in range(NL)]

    gsem = [gsem0, gsem1, gsem2]
    ssem = [ssem0, ssem1]

    def gi(j, b):
      pltpu.async_copy(x_hbm.at[src_v.at[j]], gbuf_v.at[b], gsem[b])

    def gw(j, b):
      pltpu.make_async_copy(x_hbm.at[src_v.at[j]], gbuf_v.at[b],
                            gsem[b]).wait()

    def si(j, b):
      pltpu.async_copy(sbuf_v.at[b], table.at[dst_v.at[j]], ssem[b],
                       add=True)

    def sw(j, b):
      pltpu.make_async_copy(sbuf_v.at[b], table.at[dst_v.at[j]],
                            ssem[b]).wait()

    def comp(j, gb, sb):
      def do_group(g):
        ev = ea_v[j, pl.ds(16 * g, 16)]
        for t in range(16):
          e = g * 16 + t
          ea = ev[t]
          for i in range(NL):
            r = gbuf_v[gb, e, pl.ds(16 * i, 16)]
            sbuf_v[sb, e, pl.ds(16 * i, 16)] = jnp.maximum(
                r + ea * wr[i] + br[i], 0.0)

      def egroup2(g, inner):
        do_group(2 * g)
        do_group(2 * g + 1)
        return inner
      lax.fori_loop(0, _C // 32, egroup2, None)
      do_group(_C // 16 - 1)

    for b in range(_NG):
      gi(b, b)

    def zrow(r, carry):
      for i in range(NL):
        zb_v[r, pl.ds(16 * i, 16)] = jnp.zeros((16,), jnp.float32)
      return carry
    lax.fori_loop(0, _ZR, zrow, None)
    nz = jnp.where(s == 15, 5, 8)

    def zcopy(k, carry):
      pltpu.async_copy(zb_v, table.at[pl.ds(s * _RPT + k * _ZR, _ZR)], zsem)
      return carry
    lax.fori_loop(0, nz, zcopy, None)

    def zdrain(k, carry):
      pltpu.make_async_copy(
          zb_v, table.at[pl.ds(s * _RPT + k * _ZR, _ZR)], zsem).wait()
      return carry
    lax.fori_loop(0, nz, zdrain, None)
    dst_cp.wait()
    ea_cp.wait()
    plsc.subcore_barrier()

    _ring_chunks(_CHUNKS, gi, gw, comp, si, sw, prime=False)
    plsc.subcore_barrier()

    @pl.when(s < 15)
    def _dump_full():
      pltpu.sync_copy(table.at[pl.ds(s * _RPT, _RPT)],
                      out_hbm.at[c, pl.ds(s * _RPT, _RPT)])

    @pl.when(s == 15)
    def _dump_tail():
      pltpu.sync_copy(table.at[pl.ds(15 * _RPT, _N - 15 * _RPT)],
                      out_hbm.at[c, pl.ds(15 * _RPT, _N - 15 * _RPT)])

  return sc_layer


_sc_layer_cache = {}


def _sc_layer(F):
  if F not in _sc_layer_cache:
    _sc_layer_cache[F] = _make_sc_colsplit() if F == 128 else _make_sc_layer(F)
  return _sc_layer_cache[F]


def _tc_dense1_body(x_ref, q_ref, eps_ref, W_ref, b_ref, g_ref, bt_ref,
                    o_ref):
  xa = x_ref[...]
  sc = 1.0 + eps_ref[0, 0]
  a0 = sc * xa[:, :64] + q_ref[0]
  a1 = sc * xa[:, 64:] + q_ref[1]
  W = W_ref[...]
  h = (jnp.dot(a0, W[:64], preferred_element_type=jnp.float32)
       + jnp.dot(a1, W[64:], preferred_element_type=jnp.float32)
       + b_ref[...])
  mu = jnp.mean(h, axis=0, keepdims=True)
  var = jnp.mean((h - mu) ** 2, axis=0, keepdims=True)
  o_ref[...] = jnp.maximum(
      (h - mu) * lax.rsqrt(var + 1e-5) * g_ref[...] + bt_ref[...], 0.0)


def _tc_dense1(x, q, eps, W, b, g, bt):
  H = W.shape[1]
  return pl.pallas_call(
      _tc_dense1_body,
      out_shape=jax.ShapeDtypeStruct((_N, H), jnp.float32),
  )(x, q, jnp.reshape(eps, (1, 1)), W, b.reshape(1, H), g.reshape(1, H),
    bt.reshape(1, H))


def _tc_dense_body(x_ref, p_ref, eps_ref, W_ref, b_ref, g_ref, bt_ref, o_ref):
  a = (1.0 + eps_ref[0, 0]) * x_ref[...] + p_ref[0] + p_ref[1]
  h = jnp.dot(a, W_ref[...], preferred_element_type=jnp.float32) + b_ref[...]
  mu = jnp.mean(h, axis=0, keepdims=True)
  var = jnp.mean((h - mu) ** 2, axis=0, keepdims=True)
  o_ref[...] = jnp.maximum(
      (h - mu) * lax.rsqrt(var + 1e-5) * g_ref[...] + bt_ref[...], 0.0)


def _tc_dense(x, p, eps, W, b, g, bt):
  H = W.shape[1]
  return pl.pallas_call(
      _tc_dense_body,
      out_shape=jax.ShapeDtypeStruct((_N, H), jnp.float32),
  )(x, p, jnp.reshape(eps, (1, 1)), W, b.reshape(1, H), g.reshape(1, H),
    bt.reshape(1, H))


def _tc_final_body(x_ref, p_ref, eps_ref, W_ref, b_ref, g_ref, bt_ref,
                   st_ref, Ws_ref, bs_ref, Wa1t_ref, Wa1b_ref, ba1_ref,
                   Wa2_ref, ba2_ref, o_ref):
  a = (1.0 + eps_ref[0, 0]) * x_ref[...] + p_ref[0] + p_ref[1]
  h = jnp.dot(a, W_ref[...], preferred_element_type=jnp.float32) + b_ref[...]
  mu = jnp.mean(h, axis=0, keepdims=True)
  var = jnp.mean((h - mu) ** 2, axis=0, keepdims=True)
  h = jnp.maximum(
      (h - mu) * lax.rsqrt(var + 1e-5) * g_ref[...] + bt_ref[...], 0.0)
  sx = jnp.maximum(
      jnp.dot(st_ref[...], Ws_ref[...], preferred_element_type=jnp.float32)
      + bs_ref[...], 0.0)
  u = jnp.maximum(
      jnp.dot(h, Wa1t_ref[...], preferred_element_type=jnp.float32)
      + jnp.dot(sx, Wa1b_ref[...], preferred_element_type=jnp.float32)
      + ba1_ref[...], 0.0)
  lg = (jnp.dot(u, Wa2_ref[...], preferred_element_type=jnp.float32)
        + ba2_ref[...])
  m = jnp.max(lg, axis=-1, keepdims=True)
  ex = jnp.exp(lg - m)
  o_ref[...] = ex / jnp.sum(ex, axis=-1, keepdims=True)


def _tc_final(x, p, eps, W, b, g, bt, states, Ws, bs, Wa1, ba1, Wa2, ba2):
  H = W.shape[1]
  A = Wa2.shape[1]
  return pl.pallas_call(
      _tc_final_body,
      out_shape=jax.ShapeDtypeStruct((_N, A), jnp.float32),
  )(x, p, jnp.reshape(eps, (1, 1)), W, b.reshape(1, H), g.reshape(1, H),
    bt.reshape(1, H), states.reshape(1, -1), Ws, bs.reshape(1, -1),
    Wa1[:H], Wa1[H:], ba1.reshape(1, -1), Wa2, ba2.reshape(1, -1))


def kernel(states, x, edge_attr, We1, bee1, eps1, W1, b1, g1, bt1,
           We2, bee2, eps2, W2, b2, g2, bt2, We3, bee3, eps3, W3, b3, g3, bt3,
           Ws, bs, Wa1, ba1, Wa2, ba2, edge_index, batch):
  src = edge_index[0].reshape(_NW, _CHUNKS, _C)
  dst = edge_index[1].reshape(_NW, _CHUNKS, _C)
  ea = edge_attr.reshape(_NW, _CHUNKS, _C)
  ch16 = _E // 16 // _C
  srclo16 = (2 * edge_index[0]).reshape(16, ch16, _C)
  srchi16 = (2 * edge_index[0] + 1).reshape(16, ch16, _C)
  dst16 = edge_index[1].reshape(16, ch16, _C)
  ea16 = edge_attr.reshape(16, ch16, _C)
  x2 = x.reshape(2 * _N, 64)

  q1 = _sc_layer(128)(x2, srclo16, srchi16, dst16, ea16, We1.reshape(-1),
                      bee1)
  h = _tc_dense1(x, q1, eps1, W1, b1, g1, bt1)
  p2 = _sc_layer(32)(h, src, dst, ea, We2.reshape(-1), bee2)
  h = _tc_dense(h, p2, eps2, W2, b2, g2, bt2)
  p3 = _sc_layer(32)(h, src, dst, ea, We3.reshape(-1), bee3)
  return _tc_final(h, p3, eps3, W3, b3, g3, bt3,
                   states, Ws, bs, Wa1, ba1, Wa2, ba2)


# speedup vs baseline: 1.1180x; 1.1180x over previous
import functools

import jax
import jax.numpy as jnp
from jax import lax
from jax.experimental import pallas as pl
from jax.experimental.pallas import tpu as pltpu
from jax.experimental.pallas import tpu_sc as plsc

_N = 10000
_E = 320000
_NW = 32
_C = 80
_CHUNKS = (_E // _NW) // _C
_RPT = 640
_ZR = 80


_NG = 3
_NS = 2
_UNROLL = 6


def _ring_chunks(CH, gi, gw, comp, si, sw, prime=True):
  if prime:
    for b in range(_NG):
      gi(b, b)

  def step(j, t, static):
    g, s = t % _NG, t % _NS
    gw(j, g)
    if static:
      if j >= _NS:
        sw(j - _NS, s)
    else:
      @pl.when(j >= _NS)
      def _drain_prev():
        sw(j - _NS, s)
    comp(j, g, s)
    if static:
      if j + _NG < CH:
        gi(j + _NG, g)
    else:
      @pl.when(j + _NG < CH)
      def _prefetch():
        gi(j + _NG, g)
    si(j, s)

  P = CH // _UNROLL

  def ring(i, carry):
    for t in range(_UNROLL):
      step(i * _UNROLL + t, t, False)
    return carry
  lax.fori_loop(0, P, ring, None)
  for j in range(P * _UNROLL, CH):
    step(j, j % _UNROLL, True)
  sw(CH - 2, (CH - 2) % _NS)
  sw(CH - 1, (CH - 1) % _NS)


def _make_sc_colsplit():
  NL = 4
  CH = _E // 16 // _C
  mesh = plsc.VectorSubcoreMesh(core_axis_name="c", subcore_axis_name="s")

  @functools.partial(
      pl.kernel,
      mesh=mesh,
      compiler_params=pltpu.CompilerParams(use_tc_tiling_on_sc=False),
      out_type=jax.ShapeDtypeStruct((2, _N, 64), jnp.float32),
      scratch_types=[
          pltpu.VMEM((CH, _C), jnp.int32),
          pltpu.VMEM((CH, _C), jnp.int32),
          pltpu.VMEM((CH, _C), jnp.float32),
          pltpu.VMEM((_NG, _C, 64), jnp.float32),
          pltpu.VMEM((_NS, _C, 64), jnp.float32),
          pltpu.VMEM((128,), jnp.float32),
          pltpu.VMEM((128,), jnp.float32),
          pltpu.VMEM((_ZR, 64), jnp.float32),
          pltpu.VMEM_SHARED((_N, 64), jnp.float32),
          pltpu.SemaphoreType.DMA,
          pltpu.SemaphoreType.DMA,
          pltpu.SemaphoreType.DMA,
          pltpu.SemaphoreType.DMA,
          pltpu.SemaphoreType.DMA,
          pltpu.SemaphoreType.DMA,
          pltpu.SemaphoreType.DMA,
          pltpu.SemaphoreType.DMA,
      ],
  )
  def sc_l1(x2_hbm, srclo_hbm, srchi_hbm, dst_hbm, ea_hbm, w_hbm, b_hbm,
            out_hbm, src_v, dst_v, ea_v, gbuf_v, sbuf_v, w_v, b_v, zb_v,
            table, gsem0, gsem1, gsem2, ssem0, ssem1, isem, esem, zsem):
    c = lax.axis_index("c")
    s = lax.axis_index("s")

    @pl.when(c == 0)
    def _stage_lo():
      pltpu.async_copy(srclo_hbm.at[s], src_v, isem)

    @pl.when(c == 1)
    def _stage_hi():
      pltpu.async_copy(srchi_hbm.at[s], src_v, isem)
    dst_cp = pltpu.async_copy(dst_hbm.at[s], dst_v, esem)
    ea_cp = pltpu.async_copy(ea_hbm.at[s], ea_v, esem)
    pltpu.sync_copy(w_hbm, w_v)
    pltpu.sync_copy(b_hbm, b_v)
    pltpu.make_async_copy(srclo_hbm.at[s], src_v, isem).wait()

    wr = [w_v[pl.ds(c * 64 + 16 * i, 16)] for i in range(NL)]
    br = [b_v[pl.ds(c * 64 + 16 * i, 16)] for i in range(NL)]

    gsem = [gsem0, gsem1, gsem2]
    ssem = [ssem0, ssem1]

    def gi(j, b):
      pltpu.async_copy(x2_hbm.at[src_v.at[j]], gbuf_v.at[b], gsem[b])

    def gw(j, b):
      pltpu.make_async_copy(x2_hbm.at[src_v.at[j]], gbuf_v.at[b],
                            gsem[b]).wait()

    def si(j, b):
      pltpu.async_copy(sbuf_v.at[b], table.at[dst_v.at[j]], ssem[b],
                       add=True)

    def sw(j, b):
      pltpu.make_async_copy(sbuf_v.at[b], table.at[dst_v.at[j]],
                            ssem[b]).wait()

    def comp(j, gb, sb):
      def egroup(g, inner):
        ev = ea_v[j, pl.ds(16 * g, 16)]
        for t in range(16):
          e = g * 16 + t
          ea = ev[t]
          for i in range(NL):
            r = gbuf_v[gb, e, pl.ds(16 * i, 16)]
            sbuf_v[sb, e, pl.ds(16 * i, 16)] = jnp.maximum(
                r + ea * wr[i] + br[i], 0.0)
        return inner
      lax.fori_loop(0, _C // 16, egroup, None)

    for b in range(_NG):
      gi(b, b)

    def zrow(r, carry):
      for i in range(NL):
        zb_v[r, pl.ds(16 * i, 16)] = jnp.zeros((16,), jnp.float32)
      return carry
    lax.fori_loop(0, _ZR, zrow, None)
    nz = jnp.where(s == 15, 5, 8)

    def zcopy(k, carry):
      pltpu.async_copy(zb_v, table.at[pl.ds(s * _RPT + k * _ZR, _ZR)], zsem)
      return carry
    lax.fori_loop(0, nz, zcopy, None)

    def zdrain(k, carry):
      pltpu.make_async_copy(
          zb_v, table.at[pl.ds(s * _RPT + k * _ZR, _ZR)], zsem).wait()
      return carry
    lax.fori_loop(0, nz, zdrain, None)
    dst_cp.wait()
    ea_cp.wait()
    plsc.subcore_barrier()

    _ring_chunks(CH, gi, gw, comp, si, sw, prime=False)
    plsc.subcore_barrier()

    @pl.when(s < 15)
    def _dump_full():
      pltpu.sync_copy(table.at[pl.ds(s * _RPT, _RPT)],
                      out_hbm.at[c, pl.ds(s * _RPT, _RPT)])

    @pl.when(s == 15)
    def _dump_tail():
      pltpu.sync_copy(table.at[pl.ds(15 * _RPT, _N - 15 * _RPT)],
                      out_hbm.at[c, pl.ds(15 * _RPT, _N - 15 * _RPT)])

  return sc_l1


def _make_sc_layer(F):
  NL = F // 16
  mesh = plsc.VectorSubcoreMesh(core_axis_name="c", subcore_axis_name="s")

  @functools.partial(
      pl.kernel,
      mesh=mesh,
      compiler_params=pltpu.CompilerParams(use_tc_tiling_on_sc=False),
      out_type=jax.ShapeDtypeStruct((2, _N, F), jnp.float32),
      scratch_types=[
          pltpu.VMEM((_CHUNKS, _C), jnp.int32),
          pltpu.VMEM((_CHUNKS, _C), jnp.int32),
          pltpu.VMEM((_CHUNKS, _C), jnp.float32),
          pltpu.VMEM((_NG, _C, F), jnp.float32),
          pltpu.VMEM((_NS, _C, F), jnp.float32),
          pltpu.VMEM((F,), jnp.float32),
          pltpu.VMEM((F,), jnp.float32),
          pltpu.VMEM((_ZR, F), jnp.float32),
          pltpu.VMEM_SHARED((_N, F), jnp.float32),
          pltpu.SemaphoreType.DMA,
          pltpu.SemaphoreType.DMA,
          pltpu.SemaphoreType.DMA,
          pltpu.SemaphoreType.DMA,
          pltpu.SemaphoreType.DMA,
          pltpu.SemaphoreType.DMA,
          pltpu.SemaphoreType.DMA,
          pltpu.SemaphoreType.DMA,
      ],
  )
  def sc_layer(x_hbm, src_hbm, dst_hbm, ea_hbm, w_hbm, b_hbm, out_hbm,
               src_v, dst_v, ea_v, gbuf_v, sbuf_v, w_v, b_v, zb_v, table,
               gsem0, gsem1, gsem2, ssem0, ssem1, isem, esem, zsem):
    c = lax.axis_index("c")
    s = lax.axis_index("s")
    wid = c * 16 + s

    src_cp = pltpu.async_copy(src_hbm.at[wid], src_v, isem)
    dst_cp = pltpu.async_copy(dst_hbm.at[wid], dst_v, esem)
    ea_cp = pltpu.async_copy(ea_hbm.at[wid], ea_v, esem)
    pltpu.sync_copy(w_hbm, w_v)
    pltpu.sync_copy(b_hbm, b_v)
    src_cp.wait()

    wr = [w_v[pl.ds(16 * i, 16)] for i in range(NL)]
    br = [b_v[pl.ds(16 * i, 16)] for i in range(NL)]

    gsem = [gsem0, gsem1, gsem2]
    ssem = [ssem0, ssem1]

    def gi(j, b):
      pltpu.async_copy(x_hbm.at[src_v.at[j]], gbuf_v.at[b], gsem[b])

    def gw(j, b):
      pltpu.make_async_copy(x_hbm.at[src_v.at[j]], gbuf_v.at[b],
                            gsem[b]).wait()

    def si(j, b):
      pltpu.async_copy(sbuf_v.at[b], table.at[dst_v.at[j]], ssem[b],
                       add=True)

    def sw(j, b):
      pltpu.make_async_copy(sbuf_v.at[b], table.at[dst_v.at[j]],
                            ssem[b]).wait()

    def comp(j, gb, sb):
      def egroup(g, inner):
        ev = ea_v[j, pl.ds(16 * g, 16)]
        for t in range(16):
          e = g * 16 + t
          ea = ev[t]
          for i in range(NL):
            r = gbuf_v[gb, e, pl.ds(16 * i, 16)]
            sbuf_v[sb, e, pl.ds(16 * i, 16)] = jnp.maximum(
                r + ea * wr[i] + br[i], 0.0)
        return inner
      lax.fori_loop(0, _C // 16, egroup, None)

    for b in range(_NG):
      gi(b, b)

    def zrow(r, carry):
      for i in range(NL):
        zb_v[r, pl.ds(16 * i, 16)] = jnp.zeros((16,), jnp.float32)
      return carry
    lax.fori_loop(0, _ZR, zrow, None)
    nz = jnp.where(s == 15, 5, 8)

    def zcopy(k, carry):
      pltpu.async_copy(zb_v, table.at[pl.ds(s * _RPT + k * _ZR, _ZR)], zsem)
      return carry
    lax.fori_loop(0, nz, zcopy, None)

    def zdrain(k, carry):
      pltpu.make_async_copy(
          zb_v, table.at[pl.ds(s * _RPT + k * _ZR, _ZR)], zsem).wait()
      return carry
    lax.fori_loop(0, nz, zdrain, None)
    dst_cp.wait()
    ea_cp.wait()
    plsc.subcore_barrier()

    _ring_chunks(_CHUNKS, gi, gw, comp, si, sw, prime=False)
    plsc.subcore_barrier()

    @pl.when(s < 15)
    def _dump_full():
      pltpu.sync_copy(table.at[pl.ds(s * _RPT, _RPT)],
                      out_hbm.at[c, pl.ds(s * _RPT, _RPT)])

    @pl.when(s == 15)
    def _dump_tail():
      pltpu.sync_copy(table.at[pl.ds(15 * _RPT, _N - 15 * _RPT)],
                      out_hbm.at[c, pl.ds(15 * _RPT, _N - 15 * _RPT)])

  return sc_layer


_sc_layer_cache = {}


def _sc_layer(F):
  if F not in _sc_layer_cache:
    _sc_layer_cache[F] = _make_sc_colsplit() if F == 128 else _make_sc_layer(F)
  return _sc_layer_cache[F]


def _tc_dense1_body(x_ref, q_ref, eps_ref, W_ref, b_ref, g_ref, bt_ref,
                    o_ref):
  xa = x_ref[...]
  sc = 1.0 + eps_ref[0, 0]
  a0 = sc * xa[:, :64] + q_ref[0]
  a1 = sc * xa[:, 64:] + q_ref[1]
  W = W_ref[...]
  h = (jnp.dot(a0, W[:64], preferred_element_type=jnp.float32)
       + jnp.dot(a1, W[64:], preferred_element_type=jnp.float32)
       + b_ref[...])
  mu = jnp.mean(h, axis=0, keepdims=True)
  var = jnp.mean((h - mu) ** 2, axis=0, keepdims=True)
  o_ref[...] = jnp.maximum(
      (h - mu) * lax.rsqrt(var + 1e-5) * g_ref[...] + bt_ref[...], 0.0)


def _tc_dense1(x, q, eps, W, b, g, bt):
  H = W.shape[1]
  return pl.pallas_call(
      _tc_dense1_body,
      out_shape=jax.ShapeDtypeStruct((_N, H), jnp.float32),
  )(x, q, jnp.reshape(eps, (1, 1)), W, b.reshape(1, H), g.reshape(1, H),
    bt.reshape(1, H))


def _tc_dense_body(x_ref, p_ref, eps_ref, W_ref, b_ref, g_ref, bt_ref, o_ref):
  a = (1.0 + eps_ref[0, 0]) * x_ref[...] + p_ref[0] + p_ref[1]
  h = jnp.dot(a, W_ref[...], preferred_element_type=jnp.float32) + b_ref[...]
  mu = jnp.mean(h, axis=0, keepdims=True)
  var = jnp.mean((h - mu) ** 2, axis=0, keepdims=True)
  o_ref[...] = jnp.maximum(
      (h - mu) * lax.rsqrt(var + 1e-5) * g_ref[...] + bt_ref[...], 0.0)


def _tc_dense(x, p, eps, W, b, g, bt):
  H = W.shape[1]
  return pl.pallas_call(
      _tc_dense_body,
      out_shape=jax.ShapeDtypeStruct((_N, H), jnp.float32),
  )(x, p, jnp.reshape(eps, (1, 1)), W, b.reshape(1, H), g.reshape(1, H),
    bt.reshape(1, H))


def _tc_final_body(x_ref, p_ref, eps_ref, W_ref, b_ref, g_ref, bt_ref,
                   st_ref, Ws_ref, bs_ref, Wa1t_ref, Wa1b_ref, ba1_ref,
                   Wa2_ref, ba2_ref, o_ref):
  a = (1.0 + eps_ref[0, 0]) * x_ref[...] + p_ref[0] + p_ref[1]
  h = jnp.dot(a, W_ref[...], preferred_element_type=jnp.float32) + b_ref[...]
  mu = jnp.mean(h, axis=0, keepdims=True)
  var = jnp.mean((h - mu) ** 2, axis=0, keepdims=True)
  h = jnp.maximum(
      (h - mu) * lax.rsqrt(var + 1e-5) * g_ref[...] + bt_ref[...], 0.0)
  sx = jnp.maximum(
      jnp.dot(st_ref[...], Ws_ref[...], preferred_element_type=jnp.float32)
      + bs_ref[...], 0.0)
  u = jnp.maximum(
      jnp.dot(h, Wa1t_ref[...], preferred_element_type=jnp.float32)
      + jnp.dot(sx, Wa1b_ref[...], preferred_element_type=jnp.float32)
      + ba1_ref[...], 0.0)
  lg = (jnp.dot(u, Wa2_ref[...], preferred_element_type=jnp.float32)
        + ba2_ref[...])
  m = jnp.max(lg, axis=-1, keepdims=True)
  ex = jnp.exp(lg - m)
  o_ref[...] = ex / jnp.sum(ex, axis=-1, keepdims=True)


def _tc_final(x, p, eps, W, b, g, bt, states, Ws, bs, Wa1, ba1, Wa2, ba2):
  H = W.shape[1]
  A = Wa2.shape[1]
  return pl.pallas_call(
      _tc_final_body,
      out_shape=jax.ShapeDtypeStruct((_N, A), jnp.float32),
  )(x, p, jnp.reshape(eps, (1, 1)), W, b.reshape(1, H), g.reshape(1, H),
    bt.reshape(1, H), states.reshape(1, -1), Ws, bs.reshape(1, -1),
    Wa1[:H], Wa1[H:], ba1.reshape(1, -1), Wa2, ba2.reshape(1, -1))


def kernel(states, x, edge_attr, We1, bee1, eps1, W1, b1, g1, bt1,
           We2, bee2, eps2, W2, b2, g2, bt2, We3, bee3, eps3, W3, b3, g3, bt3,
           Ws, bs, Wa1, ba1, Wa2, ba2, edge_index, batch):
  src = edge_index[0].reshape(_NW, _CHUNKS, _C)
  dst = edge_index[1].reshape(_NW, _CHUNKS, _C)
  ea = edge_attr.reshape(_NW, _CHUNKS, _C)
  ch16 = _E // 16 // _C
  srclo16 = (2 * edge_index[0]).reshape(16, ch16, _C)
  srchi16 = (2 * edge_index[0] + 1).reshape(16, ch16, _C)
  dst16 = edge_index[1].reshape(16, ch16, _C)
  ea16 = edge_attr.reshape(16, ch16, _C)
  x2 = x.reshape(2 * _N, 64)

  q1 = _sc_layer(128)(x2, srclo16, srchi16, dst16, ea16, We1.reshape(-1),
                      bee1)
  h = _tc_dense1(x, q1, eps1, W1, b1, g1, bt1)
  p2 = _sc_layer(32)(h, src, dst, ea, We2.reshape(-1), bee2)
  h = _tc_dense(h, p2, eps2, W2, b2, g2, bt2)
  p3 = _sc_layer(32)(h, src, dst, ea, We3.reshape(-1), bee3)
  return _tc_final(h, p3, eps3, W3, b3, g3, bt3,
                   states, Ws, bs, Wa1, ba1, Wa2, ba2)
